# baseline (speedup 1.0000x reference)
import functools

import jax
import jax.numpy as jnp
from jax import lax
from jax.experimental import pallas as pl
from jax.experimental.pallas import tpu as pltpu
from jax.experimental.pallas import tpu_sc as plsc

_ROWS = 4096
_COLS = 8192
_N = 500000
_NC = 2
_NS = 16
_NW = _NC * _NS
_B_PER_W = ((_N + _NW - 1) // _NW + 15) // 16 * 16
_B_PAD = _B_PER_W * _NW

_mesh = plsc.VectorSubcoreMesh(core_axis_name="c", subcore_axis_name="s")


@functools.partial(
    pl.kernel,
    out_type=jax.ShapeDtypeStruct((_B_PAD,), jnp.float32),
    mesh=_mesh,
    scratch_types=[
        pltpu.VMEM((_B_PER_W,), jnp.int32),
        pltpu.VMEM((_B_PER_W,), jnp.int32),
        pltpu.VMEM((_B_PER_W,), jnp.float32),
        pltpu.SemaphoreType.DMA,
    ],
)
def _sc_gather(xflat_hbm, rows_hbm, cols_hbm, out_hbm, rows_v, cols_v, vals_v, sem):
    wid = lax.axis_index("s") * _NC + lax.axis_index("c")
    base = wid * _B_PER_W
    pltpu.sync_copy(rows_hbm.at[pl.ds(base, _B_PER_W)], rows_v)
    pltpu.sync_copy(cols_hbm.at[pl.ds(base, _B_PER_W)], cols_v)

    def body(i, carry):
        sl = pl.ds(i * 16, 16)
        r = rows_v[sl]
        c = cols_v[sl]
        rows_v[sl] = (r % _ROWS) * _COLS + (c % _ROWS)
        return carry

    lax.fori_loop(0, _B_PER_W // 16, body, 0, unroll=4)
    flat_v = rows_v

    pltpu.async_copy(xflat_hbm.at[flat_v], vals_v, sem).wait()
    pltpu.sync_copy(vals_v, out_hbm.at[pl.ds(base, _B_PER_W)])


def kernel(x, imputed_indices):
    xflat = x.reshape(-1)
    pairs = imputed_indices.astype(jnp.int32)
    pairs = jnp.pad(pairs, ((0, _B_PAD - _N), (0, 0)))
    rows = pairs[:, 0]
    cols = pairs[:, 1]
    out = _sc_gather(xflat, rows, cols)
    return out[:_N]
